# baseline (speedup 1.0000x reference)
import functools

import jax
import jax.numpy as jnp
from jax.experimental import pallas as pl
from jax.experimental.pallas import tpu as pltpu

_BN_EPS = 1e-5
_PAD = 128
_SPLIT = 4


def _fused_block_kernel(x_ref, w_ref, s_ref, b_ref, pw_ref, b2_ref, o_ref,
                        xpad_ref, *, t, w_img, kh, kw):
    c = x_ref.shape[1]
    hw = x_ref.shape[2]
    s = pl.program_id(1)
    ph, pw_pad = kh // 2, kw // 2
    margin = w_img * ph
    w2 = t + 2 * margin

    @pl.when(s == 0)
    def _stage():
        xpad_ref[:, :_PAD] = jnp.zeros((c, _PAD), jnp.bfloat16)
        xpad_ref[:, _PAD + hw:] = jnp.zeros((c, _PAD), jnp.bfloat16)
        xpad_ref[:, _PAD:_PAD + hw] = x_ref[0].astype(jnp.bfloat16)

    win = xpad_ref[:, pl.ds(pl.multiple_of(s * t, t), t + 2 * _PAD)]

    col = jax.lax.broadcasted_iota(jnp.int32, (c, w2), 1) % w_img

    us = []
    for j in range(kw):
        dj = j - pw_pad
        u = win[:, _PAD - margin + dj:_PAD - margin + dj + w2]
        if dj < 0:
            u = jnp.where(col >= -dj, u, jnp.bfloat16(0))
        elif dj > 0:
            u = jnp.where(col < w_img - dj, u, jnp.bfloat16(0))
        us.append(u)

    acc = None
    for i in range(kh):
        v = None
        for j in range(kw):
            term = us[j] * w_ref[:, kw * i + j:kw * i + j + 1]
            v = term if v is None else v + term
        sh = w_img * i
        part = v[:, sh:sh + t]
        acc = part if acc is None else acc + part

    mid = jnp.maximum(acc * s_ref[...] + b_ref[...], jnp.bfloat16(0))
    y = jnp.dot(pw_ref[...], mid, preferred_element_type=jnp.float32)
    o_ref[0] = jnp.maximum(y + b2_ref[...], 0.0).astype(o_ref.dtype)


def kernel(x, dw_w, pw_w, bn1_gamma, bn1_beta, bn1_mean, bn1_var,
           bn2_gamma, bn2_beta, bn2_mean, bn2_var):
    n, c_in, h, w = x.shape
    kh, kw = int(dw_w.shape[2]), int(dw_w.shape[3])
    c_out = pw_w.shape[0]
    hw = h * w
    t = hw // _SPLIT

    s1 = bn1_gamma / jnp.sqrt(bn1_var + _BN_EPS)
    b1 = bn1_beta - bn1_mean * s1
    s2 = bn2_gamma / jnp.sqrt(bn2_var + _BN_EPS)
    b2 = bn2_beta - bn2_mean * s2
    pw_folded = (pw_w.reshape(c_out, c_in) * s2[:, None]).astype(jnp.bfloat16)

    x_flat = x.reshape(n, c_in, hw)
    w_taps = dw_w.reshape(c_in, kh * kw).astype(jnp.bfloat16)

    body = functools.partial(_fused_block_kernel, t=t, w_img=w, kh=kh, kw=kw)
    out_flat = pl.pallas_call(
        body,
        out_shape=jax.ShapeDtypeStruct((n, c_out, hw), x.dtype),
        grid=(n, _SPLIT),
        in_specs=[
            pl.BlockSpec((1, c_in, hw), lambda b, s: (b, 0, 0)),
            pl.BlockSpec((c_in, kh * kw), lambda b, s: (0, 0)),
            pl.BlockSpec((c_in, 1), lambda b, s: (0, 0)),
            pl.BlockSpec((c_in, 1), lambda b, s: (0, 0)),
            pl.BlockSpec((c_out, c_in), lambda b, s: (0, 0)),
            pl.BlockSpec((c_out, 1), lambda b, s: (0, 0)),
        ],
        out_specs=pl.BlockSpec((1, c_out, t), lambda b, s: (b, 0, s)),
        scratch_shapes=[pltpu.VMEM((c_in, hw + 2 * _PAD), jnp.bfloat16)],
        compiler_params=pltpu.CompilerParams(
            dimension_semantics=("parallel", "arbitrary")),
    )(x_flat, w_taps, s1.reshape(c_in, 1).astype(jnp.bfloat16),
      b1.reshape(c_in, 1).astype(jnp.bfloat16), pw_folded, b2.reshape(c_out, 1))
    return out_flat.reshape(n, c_out, h, w)


# speedup vs baseline: 1.2086x; 1.2086x over previous
import functools

import jax
import jax.numpy as jnp
from jax.experimental import pallas as pl
from jax.experimental.pallas import tpu as pltpu

_BN_EPS = 1e-5
_PAD = 128


def _fused_block_kernel(x_ref, w_ref, g1_ref, be1_ref, m1_ref, v1_ref,
                        pw_ref, g2_ref, be2_ref, m2_ref, v2_ref, o_ref,
                        xpad_ref, *, hw, w_img, kh, kw):
    c = x_ref.shape[1]
    ph, pw_pad = kh // 2, kw // 2
    w2 = hw + 2 * w_img * ph
    base = _PAD - w_img * ph

    s1 = g1_ref[...] * jax.lax.rsqrt(v1_ref[...] + _BN_EPS)
    b1 = (be1_ref[...] - m1_ref[...] * s1).astype(jnp.bfloat16)
    wt = (w_ref[...] * s1).astype(jnp.bfloat16)
    s2 = g2_ref[...] * jax.lax.rsqrt(v2_ref[...] + _BN_EPS)
    b2 = be2_ref[...] - m2_ref[...] * s2
    pwb = (pw_ref[...] * s2).astype(jnp.bfloat16)

    xpad_ref[:, :_PAD] = jnp.zeros((c, _PAD), jnp.bfloat16)
    xpad_ref[:, _PAD + hw:] = jnp.zeros((c, _PAD), jnp.bfloat16)
    xpad_ref[:, _PAD:_PAD + hw] = x_ref[0].astype(jnp.bfloat16)

    col = jax.lax.broadcasted_iota(jnp.int32, (c, w2), 1) % w_img

    us = []
    for j in range(kw):
        dj = j - pw_pad
        u = xpad_ref[:, base + dj:base + dj + w2]
        if dj < 0:
            u = jnp.where(col >= -dj, u, jnp.bfloat16(0))
        elif dj > 0:
            u = jnp.where(col < w_img - dj, u, jnp.bfloat16(0))
        us.append(u)

    acc = None
    for i in range(kh):
        v = None
        for j in range(kw):
            term = us[j] * wt[:, kw * i + j:kw * i + j + 1]
            v = term if v is None else v + term
        sh = w_img * i
        part = v[:, sh:sh + hw]
        acc = part if acc is None else acc + part

    mid = jnp.maximum(acc + b1, jnp.bfloat16(0))
    y = jnp.dot(pwb, mid, preferred_element_type=jnp.float32)
    o_ref[0] = jnp.maximum(y + b2, 0.0).astype(o_ref.dtype)


def kernel(x, dw_w, pw_w, bn1_gamma, bn1_beta, bn1_mean, bn1_var,
           bn2_gamma, bn2_beta, bn2_mean, bn2_var):
    n, c_in, h, w = x.shape
    kh, kw = int(dw_w.shape[2]), int(dw_w.shape[3])
    c_out = pw_w.shape[0]
    hw = h * w

    x_flat = x.reshape(n, c_in, hw)
    w_taps = dw_w.reshape(c_in, kh * kw)
    pw2 = pw_w.reshape(c_out, c_in)

    cvec = lambda a: a.reshape(-1, 1)
    body = functools.partial(_fused_block_kernel, hw=hw, w_img=w, kh=kh, kw=kw)
    cspec = pl.BlockSpec((c_in, 1), lambda b: (0, 0))
    cospec = pl.BlockSpec((c_out, 1), lambda b: (0, 0))
    out_flat = pl.pallas_call(
        body,
        out_shape=jax.ShapeDtypeStruct((n, c_out, hw), x.dtype),
        grid=(n,),
        in_specs=[
            pl.BlockSpec((1, c_in, hw), lambda b: (b, 0, 0)),
            pl.BlockSpec((c_in, kh * kw), lambda b: (0, 0)),
            cspec, cspec, cspec, cspec,
            pl.BlockSpec((c_out, c_in), lambda b: (0, 0)),
            cospec, cospec, cospec, cospec,
        ],
        out_specs=pl.BlockSpec((1, c_out, hw), lambda b: (b, 0, 0)),
        scratch_shapes=[pltpu.VMEM((c_in, hw + 2 * _PAD), jnp.bfloat16)],
        compiler_params=pltpu.CompilerParams(dimension_semantics=("parallel",)),
    )(x_flat, w_taps, cvec(bn1_gamma), cvec(bn1_beta), cvec(bn1_mean),
      cvec(bn1_var), pw2, cvec(bn2_gamma), cvec(bn2_beta), cvec(bn2_mean),
      cvec(bn2_var))
    return out_flat.reshape(n, c_out, h, w)


# speedup vs baseline: 1.2739x; 1.0540x over previous
import functools

import jax
import jax.numpy as jnp
from jax.experimental import pallas as pl
from jax.experimental.pallas import tpu as pltpu

_BN_EPS = 1e-5
_PAD = 128
_CHUNKS = 4


def _fused_block_kernel(x_ref, w_ref, b1_ref, pw_ref, b2_ref, o_ref,
                        xpad_ref, *, hw, w_img, kh, kw):
    c = x_ref.shape[1]
    ph, pw_pad = kh // 2, kw // 2
    margin = w_img * ph
    t = hw // _CHUNKS
    wk = t + 2 * margin

    xpad_ref[:, :_PAD] = jnp.zeros((c, _PAD), jnp.bfloat16)
    xpad_ref[:, _PAD + hw:] = jnp.zeros((c, _PAD), jnp.bfloat16)
    xpad_ref[:, _PAD:_PAD + hw] = x_ref[0].astype(jnp.bfloat16)

    b1 = b1_ref[...]
    wt = w_ref[...]
    pwb = pw_ref[...]
    b2 = b2_ref[...]

    col = jax.lax.broadcasted_iota(jnp.int32, (c, wk), 1) % w_img

    for k in range(_CHUNKS):
        base = _PAD + k * t - margin

        us = []
        for j in range(kw):
            dj = j - pw_pad
            u = xpad_ref[:, base + dj:base + dj + wk]
            if dj < 0:
                u = jnp.where(col >= -dj, u, jnp.bfloat16(0))
            elif dj > 0:
                u = jnp.where(col < w_img - dj, u, jnp.bfloat16(0))
            us.append(u)

        acc = None
        for i in range(kh):
            v = None
            for j in range(kw):
                term = us[j] * wt[:, kw * i + j:kw * i + j + 1]
                v = term if v is None else v + term
            sh = w_img * i
            part = v[:, sh:sh + t]
            acc = part if acc is None else acc + part

        mid = jnp.maximum(acc + b1, jnp.bfloat16(0))
        y = jnp.dot(pwb, mid, preferred_element_type=jnp.float32)
        o_ref[0, :, k * t:k * t + t] = jnp.maximum(y + b2, 0.0).astype(o_ref.dtype)


def kernel(x, dw_w, pw_w, bn1_gamma, bn1_beta, bn1_mean, bn1_var,
           bn2_gamma, bn2_beta, bn2_mean, bn2_var):
    n, c_in, h, w = x.shape
    kh, kw = int(dw_w.shape[2]), int(dw_w.shape[3])
    c_out = pw_w.shape[0]
    hw = h * w

    s1 = bn1_gamma / jnp.sqrt(bn1_var + _BN_EPS)
    b1 = bn1_beta - bn1_mean * s1
    s2 = bn2_gamma / jnp.sqrt(bn2_var + _BN_EPS)
    b2 = bn2_beta - bn2_mean * s2
    w_taps = (dw_w.reshape(c_in, kh * kw) * s1[:, None]).astype(jnp.bfloat16)
    pw_folded = (pw_w.reshape(c_out, c_in) * s2[:, None]).astype(jnp.bfloat16)

    x_flat = x.reshape(n, c_in, hw)

    body = functools.partial(_fused_block_kernel, hw=hw, w_img=w, kh=kh, kw=kw)
    out_flat = pl.pallas_call(
        body,
        out_shape=jax.ShapeDtypeStruct((n, c_out, hw), x.dtype),
        grid=(n,),
        in_specs=[
            pl.BlockSpec((1, c_in, hw), lambda b: (b, 0, 0)),
            pl.BlockSpec((c_in, kh * kw), lambda b: (0, 0)),
            pl.BlockSpec((c_in, 1), lambda b: (0, 0)),
            pl.BlockSpec((c_out, c_in), lambda b: (0, 0)),
            pl.BlockSpec((c_out, 1), lambda b: (0, 0)),
        ],
        out_specs=pl.BlockSpec((1, c_out, hw), lambda b: (b, 0, 0)),
        scratch_shapes=[pltpu.VMEM((c_in, hw + 2 * _PAD), jnp.bfloat16)],
        compiler_params=pltpu.CompilerParams(dimension_semantics=("parallel",)),
    )(x_flat, w_taps, b1.reshape(c_in, 1).astype(jnp.bfloat16),
      pw_folded, b2.reshape(c_out, 1))
    return out_flat.reshape(n, c_out, h, w)


# speedup vs baseline: 1.3016x; 1.0218x over previous
import functools

import jax
import jax.numpy as jnp
from jax.experimental import pallas as pl
from jax.experimental.pallas import tpu as pltpu

_BN_EPS = 1e-5
_PAD = 128
_CHUNKS = 2


def _fused_block_kernel(x_ref, w_ref, b1_ref, pw_ref, b2_ref, o_ref,
                        xpad_ref, *, hw, w_img, kh, kw):
    c = x_ref.shape[1]
    ph, pw_pad = kh // 2, kw // 2
    margin = w_img * ph
    t = hw // _CHUNKS
    wk = t + 2 * margin

    xpad_ref[:, :_PAD] = jnp.zeros((c, _PAD), jnp.bfloat16)
    xpad_ref[:, _PAD + hw:] = jnp.zeros((c, _PAD), jnp.bfloat16)
    xpad_ref[:, _PAD:_PAD + hw] = x_ref[0].astype(jnp.bfloat16)

    b1 = b1_ref[...]
    wt = w_ref[...]
    pwb = pw_ref[...]
    b2 = b2_ref[...]

    col = jax.lax.broadcasted_iota(jnp.int32, (c, wk), 1) % w_img

    for k in range(_CHUNKS):
        base = _PAD + k * t - margin

        us = []
        for j in range(kw):
            dj = j - pw_pad
            u = xpad_ref[:, base + dj:base + dj + wk]
            if dj < 0:
                u = jnp.where(col >= -dj, u, jnp.bfloat16(0))
            elif dj > 0:
                u = jnp.where(col < w_img - dj, u, jnp.bfloat16(0))
            us.append(u)

        acc = None
        for i in range(kh):
            v = None
            for j in range(kw):
                term = us[j] * wt[:, kw * i + j:kw * i + j + 1]
                v = term if v is None else v + term
            sh = w_img * i
            part = v[:, sh:sh + t]
            acc = part if acc is None else acc + part

        mid = jnp.maximum(acc + b1, jnp.bfloat16(0))
        y = jnp.dot(pwb, mid, preferred_element_type=jnp.float32)
        o_ref[0, :, k * t:k * t + t] = jnp.maximum(y + b2, 0.0).astype(o_ref.dtype)


def kernel(x, dw_w, pw_w, bn1_gamma, bn1_beta, bn1_mean, bn1_var,
           bn2_gamma, bn2_beta, bn2_mean, bn2_var):
    n, c_in, h, w = x.shape
    kh, kw = int(dw_w.shape[2]), int(dw_w.shape[3])
    c_out = pw_w.shape[0]
    hw = h * w

    s1 = bn1_gamma / jnp.sqrt(bn1_var + _BN_EPS)
    b1 = bn1_beta - bn1_mean * s1
    s2 = bn2_gamma / jnp.sqrt(bn2_var + _BN_EPS)
    b2 = bn2_beta - bn2_mean * s2
    w_taps = (dw_w.reshape(c_in, kh * kw) * s1[:, None]).astype(jnp.bfloat16)
    pw_folded = (pw_w.reshape(c_out, c_in) * s2[:, None]).astype(jnp.bfloat16)

    x_flat = x.reshape(n, c_in, hw)

    body = functools.partial(_fused_block_kernel, hw=hw, w_img=w, kh=kh, kw=kw)
    out_flat = pl.pallas_call(
        body,
        out_shape=jax.ShapeDtypeStruct((n, c_out, hw), x.dtype),
        grid=(n,),
        in_specs=[
            pl.BlockSpec((1, c_in, hw), lambda b: (b, 0, 0)),
            pl.BlockSpec((c_in, kh * kw), lambda b: (0, 0)),
            pl.BlockSpec((c_in, 1), lambda b: (0, 0)),
            pl.BlockSpec((c_out, c_in), lambda b: (0, 0)),
            pl.BlockSpec((c_out, 1), lambda b: (0, 0)),
        ],
        out_specs=pl.BlockSpec((1, c_out, hw), lambda b: (b, 0, 0)),
        scratch_shapes=[pltpu.VMEM((c_in, hw + 2 * _PAD), jnp.bfloat16)],
        compiler_params=pltpu.CompilerParams(dimension_semantics=("parallel",)),
    )(x_flat, w_taps, b1.reshape(c_in, 1).astype(jnp.bfloat16),
      pw_folded, b2.reshape(c_out, 1))
    return out_flat.reshape(n, c_out, h, w)
